# Initial kernel scaffold; baseline (speedup 1.0000x reference)
import jax
import jax.numpy as jnp
from jax.experimental import pallas as pl


def kernel(q, k, v):
    raise NotImplementedError("write your pallas kernel here")


# speedup vs baseline: 199.3881x; 199.3881x over previous
import functools

import jax
import jax.numpy as jnp
from jax import lax
from jax.experimental import pallas as pl
from jax.experimental.pallas import tpu as pltpu
from jax.experimental.pallas import tpu_sc as plsc

_L = 16


def _ce(ak, av, bk, bv):
    pred = ak <= bk
    lo_k = jnp.minimum(ak, bk)
    hi_k = jnp.maximum(ak, bk)
    lo_v = jnp.where(pred, av, bv)
    hi_v = jnp.where(pred, bv, av)
    return lo_k, lo_v, hi_k, hi_v


def _rev(x):
    return lax.rev(x, dimensions=(0,))


def _bitonic_cleanup(K, V):
    m = len(K)
    d = m // 2
    while d >= 1:
        for b in range(0, m, 2 * d):
            for i in range(b, b + d):
                K[i], V[i], K[i + d], V[i + d] = _ce(K[i], V[i], K[i + d], V[i + d])
        d //= 2
    for i in range(m):
        K[i], V[i] = plsc.sort_key_val(K[i], V[i])
    return K, V


def _merge(KA, VA, KB, VB):
    m = len(KA)
    RK = [_rev(KB[m - 1 - i]) for i in range(m)]
    RV = [_rev(VB[m - 1 - i]) for i in range(m)]
    LK, LV, HK, HV = [], [], [], []
    for i in range(m):
        lk, lv, hk, hv = _ce(KA[i], VA[i], RK[i], RV[i])
        LK.append(lk)
        LV.append(lv)
        HK.append(hk)
        HV.append(hv)
    LK, LV = _bitonic_cleanup(LK, LV)
    HK, HV = _bitonic_cleanup(HK, HV)
    return LK + HK, LV + HV


def _sort_column(K, V):
    n = len(K)
    for i in range(n):
        K[i], V[i] = plsc.sort_key_val(K[i], V[i])
    m = 1
    while m < n:
        KK, VV = [], []
        for b in range(0, n, 2 * m):
            mk, mv = _merge(K[b:b + m], V[b:b + m], K[b + m:b + 2 * m], V[b + m:b + 2 * m])
            KK += mk
            VV += mv
        K, V = KK, VV
        m *= 2
    return K, V


def _make_sc_kernel(H, S, D):
    NC, NS = 2, 16
    NV = S // _L
    assert S % _L == 0 and H % NC == 0
    cols_per_core = (H // NC) * D
    cpw = cols_per_core // NS
    assert cols_per_core % NS == 0

    mesh = plsc.VectorSubcoreMesh(
        core_axis_name="c", subcore_axis_name="s", num_cores=NC, num_subcores=NS
    )

    @functools.partial(
        pl.kernel,
        out_type=jax.ShapeDtypeStruct((H * D, S), jnp.float32),
        mesh=mesh,
        scratch_types=[
            pltpu.VMEM((S,), jnp.float32),
            pltpu.VMEM((cpw * S,), jnp.float32),
            pltpu.VMEM((S,), jnp.int32),
            pltpu.VMEM((S,), jnp.float32),
            pltpu.VMEM_SHARED((cols_per_core, S), jnp.int32),
        ],
        compiler_params=pltpu.CompilerParams(needs_layout_passes=False),
    )
    def kern(v_ref, out_ref, col_buf, sv_all, idx_buf, out_buf, pi_sh):
        c = lax.axis_index("c")
        s_id = lax.axis_index("s")
        base_local = s_id * cpw
        core_base = c * cols_per_core

        def phase1(j, carry):
            lcol = base_local + j
            gcol = core_base + lcol
            pltpu.sync_copy(v_ref.at[gcol], col_buf)
            K = [col_buf[pl.ds(_L * i, _L)] for i in range(NV)]
            iota = lax.iota(jnp.int32, _L)
            V = [iota + _L * i for i in range(NV)]
            K, V = _sort_column(K, V)
            for i in range(NV):
                sv_all[pl.ds(j * S + _L * i, _L)] = K[i]
                idx_buf[pl.ds(_L * i, _L)] = V[i]
            pltpu.sync_copy(idx_buf, pi_sh.at[lcol])
            return carry

        lax.fori_loop(0, cpw, phase1, 0)
        plsc.subcore_barrier()

        def phase2(j, carry):
            lcol = base_local + j
            gcol = core_base + lcol
            h_loc = lax.div(lcol, D)
            d = lax.rem(lcol, D)
            for kk in (1, 2, 3):
                nb = h_loc * D + lax.rem(d + kk, D)
                pltpu.sync_copy(pi_sh.at[nb], idx_buf)
                for r in range(NV):
                    idxv = idx_buf[pl.ds(_L * r, _L)]
                    val = sv_all[pl.ds(j * S + _L * r, _L)]
                    if kk == 1:
                        plsc.store_scatter(out_buf, [idxv], val)
                    else:
                        plsc.addupdate_scatter(out_buf, [idxv], val)
            pltpu.sync_copy(out_buf, out_ref.at[gcol])
            return carry

        lax.fori_loop(0, cpw, phase2, 0)

    return kern


def kernel(q, k, v):
    del q, k
    B, H, S, D = v.shape
    vT = jnp.transpose(v, (0, 1, 3, 2)).reshape(B * H * D, S)
    kern = _make_sc_kernel(B * H, S, D)
    outT = kern(vT)
    out = jnp.transpose(outT.reshape(B, H, D, S), (0, 1, 3, 2))
    return (out, out)
